# baseline (speedup 1.0000x reference)
import functools

import jax
import jax.numpy as jnp
from jax import lax
from jax.experimental import pallas as pl
from jax.experimental.pallas import tpu as pltpu
from jax.experimental.pallas import tpu_sc as plsc

B = 16384
D = 64
NC = 2
NS = 16
NW = NC * NS
BPW = B // NW
CH = 64
NCH = BPW // CH
L = 16
PAD = L + 1

_mesh = plsc.VectorSubcoreMesh(core_axis_name="c", subcore_axis_name="s")


@functools.partial(
    pl.kernel,
    out_type=jax.ShapeDtypeStruct((B,), jnp.float32),
    mesh=_mesh,
    compiler_params=pltpu.CompilerParams(
        needs_layout_passes=False, use_tc_tiling_on_sc=True
    ),
    scratch_types=[
        pltpu.VMEM((BPW,), jnp.int32),
        pltpu.VMEM((BPW,), jnp.int32),
        pltpu.VMEM((2, CH, D), jnp.float32),
        pltpu.VMEM((2, CH, D), jnp.float32),
        pltpu.VMEM((L * PAD,), jnp.float32),
        pltpu.VMEM((BPW,), jnp.float32),
        pltpu.SemaphoreType.DMA,
        pltpu.SemaphoreType.DMA,
        pltpu.SemaphoreType.DMA,
        pltpu.SemaphoreType.DMA,
    ],
)
def _sc_dual_gather_dot(idx_hbm, user_hbm, item_hbm, out_hbm,
                        uix, iix, urows, irows, tmat, outv,
                        usem0, usem1, isem0, isem1):
    wid = lax.axis_index("s") * NC + lax.axis_index("c")
    base = wid * BPW

    pltpu.sync_copy(idx_hbm.at[0, pl.ds(base, BPW)], uix)
    pltpu.sync_copy(idx_hbm.at[1, pl.ds(base, BPW)], iix)

    usems = [usem0, usem1]
    isems = [isem0, isem1]

    def fire_group(q, buf, g):
        k0 = g * L
        uvec = uix[pl.ds(q * CH + k0, L)]
        ivec = iix[pl.ds(q * CH + k0, L)]
        for j in range(L):
            pltpu.async_copy(user_hbm.at[pl.ds(uvec[j], 1), :],
                             urows.at[buf, pl.ds(k0 + j, 1), :], usems[buf])
            pltpu.async_copy(item_hbm.at[pl.ds(ivec[j], 1), :],
                             irows.at[buf, pl.ds(k0 + j, 1), :], isems[buf])

    def drain_group(buf):
        pltpu.make_async_copy(user_hbm.at[pl.ds(0, L), :],
                              urows.at[buf, pl.ds(0, L), :],
                              usems[buf]).wait()
        pltpu.make_async_copy(item_hbm.at[pl.ds(0, L), :],
                              irows.at[buf, pl.ds(0, L), :],
                              isems[buf]).wait()

    NG = CH // L

    def fire(q):
        buf = q % 2

        def body(g, _):
            fire_group(q, buf, g)
            return 0

        lax.fori_loop(0, NG, body, 0)
        return buf

    def drain_tail(buf):
        for _ in range(NG):
            drain_group(buf)

    iota = lax.iota(jnp.int32, L)
    gather_idx = [iota * PAD + l for l in range(L)]

    def compute_chunk(q):
        buf = q % 2

        def block_body(blk, _):
            rbase = blk * L
            for j in range(L):
                b = rbase + j
                s = (urows[buf, b, pl.ds(0, L)]
                     * irows[buf, b, pl.ds(0, L)])
                for d0 in range(L, D, L):
                    s = s + (urows[buf, b, pl.ds(d0, L)]
                             * irows[buf, b, pl.ds(d0, L)])
                tmat[pl.ds(j * PAD, L)] = s
            acc = plsc.load_gather(tmat, [gather_idx[0]])
            for l in range(1, L):
                acc = acc + plsc.load_gather(tmat, [gather_idx[l]])
            outv[pl.ds(q * CH + rbase, L)] = acc
            return 0

        lax.fori_loop(0, CH // L, block_body, 0)

    buf = fire(0)
    for q in range(NCH):
        nxt = fire(q + 1) if q + 1 < NCH else None
        drain_tail(buf)
        compute_chunk(q)
        buf = nxt

    pltpu.sync_copy(outv, out_hbm.at[pl.ds(base, BPW)])


def kernel(inputs, user_table, item_table):
    return _sc_dual_gather_dot(inputs.T, user_table, item_table)
